# baseline (speedup 1.0000x reference)
import functools

import jax
import jax.numpy as jnp
from jax import lax
from jax.experimental import pallas as pl
from jax.experimental.pallas import tpu as pltpu
from jax.experimental.pallas import tpu_sc as plsc

N = 10000
E = 320000
H = 128

NC = 2
NS = 16
NW = NC * NS
EPT = E // NW
CHP = 64
EPTP = 10240
NCHP = EPTP // CHP
NPAD = 10240
RPW = NPAD // NS

def _wid():
  return lax.axis_index("s") * NC + lax.axis_index("c")


@functools.lru_cache(maxsize=None)
def _mesh():
  return plsc.VectorSubcoreMesh(
      core_axis_name="c", subcore_axis_name="s",
      num_cores=NC, num_subcores=NS)


_SC_PARAMS = pltpu.CompilerParams(needs_layout_passes=False)
_SC_PARAMS_NT = pltpu.CompilerParams(
    needs_layout_passes=False, use_tc_tiling_on_sc=False)


@functools.lru_cache(maxsize=None)
def _sc_deg_kernel():
  return pl.kernel(
      _sc_deg_body,
      out_type=jax.ShapeDtypeStruct((NW, N), jnp.float32),
      mesh=_mesh(),
      compiler_params=_SC_PARAMS,
      scratch_types=[
          pltpu.VMEM((EPT,), jnp.int32),
          pltpu.VMEM((EPT,), jnp.float32),
          pltpu.VMEM((N,), jnp.float32),
      ],
  )


def _sc_deg_body(dst_hbm, w_hbm, out_hbm, dstv, wv, acc):
  base = _wid() * EPT
  pltpu.sync_copy(dst_hbm.at[pl.ds(base, EPT)], dstv)
  pltpu.sync_copy(w_hbm.at[pl.ds(base, EPT)], wv)

  def zero(i, _):
    acc[pl.ds(i * 16, 16)] = jnp.zeros((16,), jnp.float32)
    return 0
  lax.fori_loop(0, N // 16, zero, 0)

  def body(i, _):
    sl = pl.ds(i * 16, 16)
    plsc.addupdate_scatter(acc, [dstv[sl]], wv[sl])
    return 0
  lax.fori_loop(0, EPT // 16, body, 0)

  pltpu.sync_copy(acc, out_hbm.at[_wid()])


@functools.lru_cache(maxsize=None)
def _sc_sprop_kernel():
  return pl.kernel(
      _sc_sprop_body,
      out_type=jax.ShapeDtypeStruct((NW, N), jnp.float32),
      mesh=_mesh(),
      compiler_params=_SC_PARAMS,
      scratch_types=[
          pltpu.VMEM((EPT,), jnp.int32),
          pltpu.VMEM((EPT,), jnp.int32),
          pltpu.VMEM((EPT,), jnp.float32),
          pltpu.VMEM((N,), jnp.float32),
          pltpu.VMEM((N,), jnp.float32),
      ],
  )


def _sc_sprop_body(src_hbm, dst_hbm, w_hbm, v_hbm, out_hbm, srcv, dstv, wv, vv, acc):
  base = _wid() * EPT
  pltpu.sync_copy(src_hbm.at[pl.ds(base, EPT)], srcv)
  pltpu.sync_copy(dst_hbm.at[pl.ds(base, EPT)], dstv)
  pltpu.sync_copy(w_hbm.at[pl.ds(base, EPT)], wv)
  pltpu.sync_copy(v_hbm, vv)

  def zero(i, _):
    acc[pl.ds(i * 16, 16)] = jnp.zeros((16,), jnp.float32)
    return 0
  lax.fori_loop(0, N // 16, zero, 0)

  def body(i, _):
    sl = pl.ds(i * 16, 16)
    vals = plsc.load_gather(vv, [srcv[sl]])
    plsc.addupdate_scatter(acc, [dstv[sl]], vals * wv[sl])
    return 0
  lax.fori_loop(0, EPT // 16, body, 0)

  pltpu.sync_copy(acc, out_hbm.at[_wid()])


@functools.lru_cache(maxsize=None)
def _sc_hprop_kernel():
  return pl.kernel(
      _sc_hprop_body,
      out_type=jax.ShapeDtypeStruct((NC, NPAD, H), jnp.float32),
      mesh=_mesh(),
      compiler_params=_SC_PARAMS_NT,
      scratch_types=[
          pltpu.VMEM((2, CHP), jnp.int32),
          pltpu.VMEM((2, CHP), jnp.int32),
          pltpu.VMEM((CHP,), jnp.int32),
          pltpu.VMEM((CHP,), jnp.int32),
          pltpu.VMEM((CHP,), jnp.int32),
          pltpu.VMEM((CHP,), jnp.int32),
          pltpu.VMEM((CHP,), jnp.float32),
          pltpu.VMEM((CHP,), jnp.float32),
          pltpu.VMEM((CHP, H // 2), jnp.int32),
          pltpu.VMEM((CHP, H // 2), jnp.int32),
          pltpu.VMEM((CHP, H), jnp.float32),
          pltpu.VMEM((CHP, H), jnp.float32),
          pltpu.VMEM_SHARED((NPAD, H), jnp.float32),
          pltpu.SemaphoreType.DMA,
          pltpu.SemaphoreType.DMA,
          pltpu.SemaphoreType.DMA,
          pltpu.SemaphoreType.DMA,
          pltpu.SemaphoreType.DMA,
          pltpu.SemaphoreType.DMA,
      ],
  )


def _sc_hprop_body(e4_hbm, hs_hbm, out_hbm,
                   ebuf0, ebuf1, sidx0, sidx1, didx0, didx1, wbuf0, wbuf1,
                   rgi0, rgi1, rsf0, rsf1, accsh,
                   esem0, esem1, gsem0, gsem1, ssem0, ssem1):
  c = lax.axis_index("c")
  s = lax.axis_index("s")
  wid = _wid()
  ebuf = (ebuf0, ebuf1)
  sidx = (sidx0, sidx1)
  didx = (didx0, didx1)
  wbuf = (wbuf0, wbuf1)
  rgi = (rgi0, rgi1)
  rsf = (rsf0, rsf1)
  esem = (esem0, esem1)
  gsem = (gsem0, gsem1)
  ssem = (ssem0, ssem1)

  def zfill(i, _):
    for j in range(H // 16):
      rsf0[i, pl.ds(j * 16, 16)] = jnp.zeros((16,), jnp.float32)
    return 0
  lax.fori_loop(0, CHP, zfill, 0)
  for k in range(RPW // CHP):
    pltpu.sync_copy(rsf0, accsh.at[pl.ds(s * RPW + k * CHP, CHP)])
  plsc.subcore_barrier()

  def erec_start(j, b):
    pltpu.async_copy(e4_hbm.at[wid, j], ebuf[b], esem[b])

  def erec_wait(b):
    pltpu.make_async_copy(e4_hbm.at[wid, 0], ebuf[b], esem[b]).wait()

  def unpack(b):
    for t in range(CHP // 16):
      sl = pl.ds(t * 16, 16)
      p = ebuf[b][0, sl]
      sidx[b][sl] = lax.shift_right_logical(p, 16)
      didx[b][sl] = lax.bitwise_and(p, 0xFFFF)
      wbuf[b][sl] = plsc.bitcast(ebuf[b][1, sl], jnp.float32)

  def gather_start(b):
    pltpu.async_copy(hs_hbm.at[sidx[b]], rgi[b], gsem[b])

  def gather_wait(b):
    pltpu.make_async_copy(hs_hbm.at[sidx[b]], rgi[b], gsem[b]).wait()

  def scatter_start(b):
    pltpu.async_copy(rsf[b], accsh.at[didx[b]], ssem[b], add=True)

  def scatter_wait(b):
    pltpu.make_async_copy(rsf[b], accsh.at[didx[b]], ssem[b]).wait()

  UNROLL = 4
  HIMASK = jnp.int32(-65536)

  def scale(b):
    rg = rgi[b]
    rs = rsf[b]
    wv = wbuf[b]

    def srow(r, _):
      for u in range(UNROLL):
        row = r * UNROLL + u
        wsc = plsc.load_gather(wv, [jnp.zeros((16,), jnp.int32) + row])
        for t in range(H // 32):
          sl = pl.ds(t * 16, 16)
          x = rg[row, sl]
          lo = plsc.bitcast(lax.shift_left(x, 16), jnp.float32)
          hi = plsc.bitcast(lax.bitwise_and(x, HIMASK), jnp.float32)
          rs[row, sl] = lo * wsc
          rs[row, pl.ds(H // 2 + t * 16, 16)] = hi * wsc
      return 0
    lax.fori_loop(0, CHP // UNROLL, srow, 0)

  erec_start(0, 0)
  erec_wait(0)
  unpack(0)
  gather_start(0)
  erec_start(1, 1)

  def pair(i, _):
    for b in range(2):
      j = 2 * i + b
      nxt = 1 - b
      @pl.when(i < NCHP // 2 - 1)
      def _():
        erec_start(j + 2, b)
      if b == 0:
        @pl.when(i >= 1)
        def _():
          scatter_wait(1)
      else:
        scatter_wait(0)

      @pl.when(j + 1 < NCHP)
      def _():
        erec_wait(nxt)
        unpack(nxt)
        gather_start(nxt)

      gather_wait(b)
      scale(b)
      scatter_start(b)
    return 0
  lax.fori_loop(0, NCHP // 2, pair, 0)

  scatter_wait(1)
  plsc.subcore_barrier()
  for k in range(RPW // CHP):
    r0 = s * RPW + k * CHP
    pltpu.sync_copy(accsh.at[pl.ds(r0, CHP)], rsf0)
    pltpu.sync_copy(rsf0, out_hbm.at[c, pl.ds(r0, CHP)])


def _tc_deg_body(parts_ref, x_ref, dinv_ref, xs_ref):
  deg = 1.0 + jnp.sum(parts_ref[...], axis=0)
  dinv = jnp.where(deg > 0, lax.rsqrt(deg), 0.0)[:, None]
  dinv_ref[...] = dinv
  xs_ref[...] = dinv * x_ref[...]


def _tc_deg(parts, x):
  return pl.pallas_call(
      _tc_deg_body,
      out_shape=[jax.ShapeDtypeStruct((N, 1), jnp.float32),
                 jax.ShapeDtypeStruct((N, 1), jnp.float32)],
  )(parts, x)


def _tc_layer1_body(parts_ref, xs_ref, dinv_ref, w1_ref, b1_ref, hs1_ref):
  p = jnp.sum(parts_ref[...], axis=0)[:, None]
  px = dinv_ref[...] * (p + xs_ref[...])
  h1 = jax.nn.relu(px * w1_ref[...] + b1_ref[...][None, :])
  hs1_ref[...] = dinv_ref[...] * h1


def _tc_layer1(parts, xs, dinv, W1, b1):
  return pl.pallas_call(
      _tc_layer1_body,
      out_shape=jax.ShapeDtypeStruct((N, H), jnp.float32),
  )(parts, xs, dinv, W1, b1)


_RB = 2000


def _tc_layer_body(q_ref, hs_ref, dinv_ref, w_ref, b_ref, out_ref):
  ph = dinv_ref[...] * (q_ref[0] + q_ref[1] + hs_ref[...])
  h = jax.nn.relu(
      jnp.dot(ph, w_ref[...], preferred_element_type=jnp.float32)
      + b_ref[...][None, :])
  out_ref[...] = dinv_ref[...] * h


def _tc_layer(q, hs, dinv, W, b):
  return pl.pallas_call(
      _tc_layer_body,
      grid=(N // _RB,),
      in_specs=[
          pl.BlockSpec((NC, _RB, H), lambda i: (0, i, 0)),
          pl.BlockSpec((_RB, H), lambda i: (i, 0)),
          pl.BlockSpec((_RB, 1), lambda i: (i, 0)),
          pl.BlockSpec((H, H), lambda i: (0, 0)),
          pl.BlockSpec((H,), lambda i: (0,)),
      ],
      out_specs=pl.BlockSpec((_RB, H), lambda i: (i, 0)),
      out_shape=jax.ShapeDtypeStruct((N, H), jnp.float32),
  )(q, hs, dinv, W, b)


def _tc_layer34_body(q_ref, hs_ref, dinv_ref, w3_ref, b3_ref, w4_ref, ts_ref):
  ph = dinv_ref[...] * (q_ref[0] + q_ref[1] + hs_ref[...])
  h3 = jax.nn.relu(
      jnp.dot(ph, w3_ref[...], preferred_element_type=jnp.float32)
      + b3_ref[...][None, :])
  ts_ref[...] = dinv_ref[...] * jnp.dot(
      h3, w4_ref[...], preferred_element_type=jnp.float32)


def _tc_layer34(q, hs, dinv, W3, b3, W4):
  return pl.pallas_call(
      _tc_layer34_body,
      grid=(N // _RB,),
      in_specs=[
          pl.BlockSpec((NC, _RB, H), lambda i: (0, i, 0)),
          pl.BlockSpec((_RB, H), lambda i: (i, 0)),
          pl.BlockSpec((_RB, 1), lambda i: (i, 0)),
          pl.BlockSpec((H, H), lambda i: (0, 0)),
          pl.BlockSpec((H,), lambda i: (0,)),
          pl.BlockSpec((H, 1), lambda i: (0, 0)),
      ],
      out_specs=pl.BlockSpec((_RB, 1), lambda i: (i, 0)),
      out_shape=jax.ShapeDtypeStruct((N, 1), jnp.float32),
  )(q, hs, dinv, W3, b3, W4)


def _tc_final_body(parts_ref, ts_ref, dinv_ref, b4_ref, out_ref):
  o = jnp.sum(parts_ref[...], axis=0)[:, None]
  out_ref[...] = dinv_ref[...] * (o + ts_ref[...]) + b4_ref[0]


def _tc_final(parts, ts, dinv, b4):
  return pl.pallas_call(
      _tc_final_body,
      out_shape=jax.ShapeDtypeStruct((N, 1), jnp.float32),
  )(parts, ts, dinv, b4)


def _pack_bf16(hs):
  lob = lax.bitcast_convert_type(
      hs[:, :H // 2].astype(jnp.bfloat16), jnp.uint16).astype(jnp.uint32)
  hib = lax.bitcast_convert_type(
      hs[:, H // 2:].astype(jnp.bfloat16), jnp.uint16).astype(jnp.uint32)
  return lax.bitcast_convert_type((hib << 16) | lob, jnp.int32)


def kernel(x, edge_index, edge_weight, W1, b1, W2, b2, W3, b3, W4, b4):
  src = edge_index[0]
  dst = edge_index[1]
  w = edge_weight

  degp = _sc_deg_kernel()(dst, w)
  dinv, xs = _tc_deg(degp, x)
  pparts = _sc_sprop_kernel()(src, dst, w, xs[:, 0])
  hs1 = _tc_layer1(pparts, xs, dinv, W1, b1)
  pad = ((0, 0), (0, EPTP - EPT))
  srcp = jnp.pad(src.reshape(NW, EPT), pad)
  dstp = jnp.pad(dst.reshape(NW, EPT), pad, constant_values=NPAD - 1)
  wp = jnp.pad(w.reshape(NW, EPT), pad)
  packed = (srcp << 16) | dstp
  wbits = lax.bitcast_convert_type(wp, jnp.int32)
  e4 = jnp.stack([packed.reshape(NW, NCHP, CHP),
                  wbits.reshape(NW, NCHP, CHP)], axis=2)
  q2 = _sc_hprop_kernel()(e4, _pack_bf16(hs1))
  hs2 = _tc_layer(q2, hs1, dinv, W2, b2)
  q3 = _sc_hprop_kernel()(e4, _pack_bf16(hs2))
  ts = _tc_layer34(q3, hs2, dinv, W3, b3, W4)
  oparts = _sc_sprop_kernel()(src, dst, w, ts[:, 0])
  return _tc_final(oparts, ts, dinv, b4)


# speedup vs baseline: 1.6991x; 1.6991x over previous
import functools

import jax
import jax.numpy as jnp
from jax import lax
from jax.experimental import pallas as pl
from jax.experimental.pallas import tpu as pltpu
from jax.experimental.pallas import tpu_sc as plsc

N = 10000
E = 320000
H = 128

NC = 2
NS = 16
NW = NC * NS
EPT = E // NW
CHP = 64
EPTP = 10240
NCHP = EPTP // CHP
NPAD = 10240
RPW = NPAD // NS

def _wid():
  return lax.axis_index("s") * NC + lax.axis_index("c")


@functools.lru_cache(maxsize=None)
def _mesh():
  return plsc.VectorSubcoreMesh(
      core_axis_name="c", subcore_axis_name="s",
      num_cores=NC, num_subcores=NS)


_SC_PARAMS = pltpu.CompilerParams(needs_layout_passes=False)
_SC_PARAMS_NT = pltpu.CompilerParams(
    needs_layout_passes=False, use_tc_tiling_on_sc=False)


@functools.lru_cache(maxsize=None)
def _sc_deg_kernel():
  return pl.kernel(
      _sc_deg_body,
      out_type=jax.ShapeDtypeStruct((NW, N), jnp.float32),
      mesh=_mesh(),
      compiler_params=_SC_PARAMS,
      scratch_types=[
          pltpu.VMEM((EPT,), jnp.int32),
          pltpu.VMEM((EPT,), jnp.float32),
          pltpu.VMEM((N,), jnp.float32),
      ],
  )


def _sc_deg_body(dst_hbm, w_hbm, out_hbm, dstv, wv, acc):
  base = _wid() * EPT
  pltpu.sync_copy(dst_hbm.at[pl.ds(base, EPT)], dstv)
  pltpu.sync_copy(w_hbm.at[pl.ds(base, EPT)], wv)

  def zero(i, _):
    acc[pl.ds(i * 16, 16)] = jnp.zeros((16,), jnp.float32)
    return 0
  lax.fori_loop(0, N // 16, zero, 0)

  def body(i, _):
    sl = pl.ds(i * 16, 16)
    plsc.addupdate_scatter(acc, [dstv[sl]], wv[sl])
    return 0
  lax.fori_loop(0, EPT // 16, body, 0)

  pltpu.sync_copy(acc, out_hbm.at[_wid()])


@functools.lru_cache(maxsize=None)
def _sc_sprop_kernel():
  return pl.kernel(
      _sc_sprop_body,
      out_type=jax.ShapeDtypeStruct((NW, N), jnp.float32),
      mesh=_mesh(),
      compiler_params=_SC_PARAMS,
      scratch_types=[
          pltpu.VMEM((EPT,), jnp.int32),
          pltpu.VMEM((EPT,), jnp.int32),
          pltpu.VMEM((EPT,), jnp.float32),
          pltpu.VMEM((N,), jnp.float32),
          pltpu.VMEM((N,), jnp.float32),
      ],
  )


def _sc_sprop_body(src_hbm, dst_hbm, w_hbm, v_hbm, out_hbm, srcv, dstv, wv, vv, acc):
  base = _wid() * EPT
  pltpu.sync_copy(src_hbm.at[pl.ds(base, EPT)], srcv)
  pltpu.sync_copy(dst_hbm.at[pl.ds(base, EPT)], dstv)
  pltpu.sync_copy(w_hbm.at[pl.ds(base, EPT)], wv)
  pltpu.sync_copy(v_hbm, vv)

  def zero(i, _):
    acc[pl.ds(i * 16, 16)] = jnp.zeros((16,), jnp.float32)
    return 0
  lax.fori_loop(0, N // 16, zero, 0)

  def body(i, _):
    sl = pl.ds(i * 16, 16)
    vals = plsc.load_gather(vv, [srcv[sl]])
    plsc.addupdate_scatter(acc, [dstv[sl]], vals * wv[sl])
    return 0
  lax.fori_loop(0, EPT // 16, body, 0)

  pltpu.sync_copy(acc, out_hbm.at[_wid()])


CHT = 16
NCHT = EPTP // CHT


@functools.lru_cache(maxsize=None)
def _sc_hprop_kernel():
  return pl.kernel(
      _sc_hprop_body,
      out_type=jax.ShapeDtypeStruct((NC, NPAD, H), jnp.float32),
      mesh=_mesh(),
      compiler_params=_SC_PARAMS_NT,
      scratch_types=[
          pltpu.VMEM((2, CHT), jnp.int32),
          pltpu.VMEM((2, CHT), jnp.int32),
          pltpu.VMEM((CHT,), jnp.int32),
          pltpu.VMEM((CHT,), jnp.int32),
          pltpu.VMEM((CHT,), jnp.int32),
          pltpu.VMEM((CHT,), jnp.int32),
          pltpu.VMEM((CHT, H // 2), jnp.int32),
          pltpu.VMEM((CHT, H // 2), jnp.int32),
          pltpu.VMEM((CHT, H), jnp.float32),
          pltpu.VMEM((CHT, H), jnp.float32),
          pltpu.VMEM_SHARED((NPAD, H // 2), jnp.int32),
          pltpu.VMEM_SHARED((NPAD, H), jnp.float32),
          pltpu.SemaphoreType.DMA,
          pltpu.SemaphoreType.DMA,
          pltpu.SemaphoreType.DMA,
          pltpu.SemaphoreType.DMA,
          pltpu.SemaphoreType.DMA,
          pltpu.SemaphoreType.DMA,
      ],
  )


def _sc_hprop_body(e4_hbm, hsp_hbm, out_hbm,
                   ebuf0, ebuf1, sidx0, sidx1, didx0, didx1,
                   rgi0, rgi1, rsf0, rsf1, tabsh, accsh,
                   esem0, esem1, gsem0, gsem1, ssem0, ssem1):
  c = lax.axis_index("c")
  s = lax.axis_index("s")
  wid = _wid()
  ebuf = (ebuf0, ebuf1)
  sidx = (sidx0, sidx1)
  didx = (didx0, didx1)
  rgi = (rgi0, rgi1)
  rsf = (rsf0, rsf1)
  esem = (esem0, esem1)
  gsem = (gsem0, gsem1)
  ssem = (ssem0, ssem1)

  pltpu.sync_copy(hsp_hbm.at[pl.ds(s * RPW, RPW)], tabsh.at[pl.ds(s * RPW, RPW)])
  for i in range(CHT):
    for j in range(H // 16):
      rsf0[i, pl.ds(j * 16, 16)] = jnp.zeros((16,), jnp.float32)
  def zcopy(k, _):
    pltpu.sync_copy(rsf0, accsh.at[pl.ds(s * RPW + k * CHT, CHT)])
    return 0
  lax.fori_loop(0, RPW // CHT, zcopy, 0)
  plsc.subcore_barrier()

  def erec_start(j, b):
    pltpu.async_copy(e4_hbm.at[wid, j], ebuf[b], esem[b])

  def erec_wait(b):
    pltpu.make_async_copy(e4_hbm.at[wid, 0], ebuf[b], esem[b]).wait()

  def unpack(b):
    p = ebuf[b][0, pl.ds(0, CHT)]
    sidx[b][pl.ds(0, CHT)] = lax.shift_right_logical(p, 16)
    didx[b][pl.ds(0, CHT)] = lax.bitwise_and(p, 0xFFFF)

  def gather_start(b):
    pltpu.async_copy(tabsh.at[sidx[b]], rgi[b], gsem[b])

  def gather_wait(b):
    pltpu.make_async_copy(tabsh.at[sidx[b]], rgi[b], gsem[b]).wait()

  def scatter_start(b):
    pltpu.async_copy(rsf[b], accsh.at[didx[b]], ssem[b], add=True)

  def scatter_wait(b):
    pltpu.make_async_copy(rsf[b], accsh.at[didx[b]], ssem[b]).wait()

  HIMASK = jnp.int32(-65536)

  def scale(b):
    rg = rgi[b]
    rs = rsf[b]
    wv = plsc.bitcast(ebuf[b][1, pl.ds(0, CHT)], jnp.float32)
    for row in range(CHT):
      for t in range(H // 32):
        sl = pl.ds(t * 16, 16)
        x = rg[row, sl]
        lo = plsc.bitcast(lax.shift_left(x, 16), jnp.float32)
        hi = plsc.bitcast(lax.bitwise_and(x, HIMASK), jnp.float32)
        rs[row, sl] = lo * wv[row]
        rs[row, pl.ds(H // 2 + t * 16, 16)] = hi * wv[row]

  erec_start(0, 0)
  erec_wait(0)
  unpack(0)
  gather_start(0)
  erec_start(1, 1)

  def pair(i, _):
    for b in range(2):
      j = 2 * i + b
      nxt = 1 - b
      @pl.when(i < NCHT // 2 - 1)
      def _():
        erec_start(j + 2, b)
      if b == 0:
        @pl.when(i >= 1)
        def _():
          scatter_wait(1)
      else:
        scatter_wait(0)

      @pl.when(j + 1 < NCHT)
      def _():
        erec_wait(nxt)
        unpack(nxt)
        gather_start(nxt)

      gather_wait(b)
      scale(b)
      scatter_start(b)
    return 0
  lax.fori_loop(0, NCHT // 2, pair, 0)

  scatter_wait(1)
  plsc.subcore_barrier()
  pltpu.sync_copy(accsh.at[pl.ds(s * RPW, RPW)], out_hbm.at[c, pl.ds(s * RPW, RPW)])


def _tc_deg_body(parts_ref, x_ref, dinv_ref, xs_ref):
  deg = 1.0 + jnp.sum(parts_ref[...], axis=0)
  dinv = jnp.where(deg > 0, lax.rsqrt(deg), 0.0)[:, None]
  dinv_ref[...] = dinv
  xs_ref[...] = dinv * x_ref[...]


def _tc_deg(parts, x):
  return pl.pallas_call(
      _tc_deg_body,
      out_shape=[jax.ShapeDtypeStruct((N, 1), jnp.float32),
                 jax.ShapeDtypeStruct((N, 1), jnp.float32)],
  )(parts, x)


def _tc_layer1_body(parts_ref, xs_ref, dinv_ref, w1_ref, b1_ref, hs1_ref):
  p = jnp.sum(parts_ref[...], axis=0)[:, None]
  px = dinv_ref[...] * (p + xs_ref[...])
  h1 = jax.nn.relu(px * w1_ref[...] + b1_ref[...][None, :])
  hs1_ref[...] = dinv_ref[...] * h1


def _tc_layer1(parts, xs, dinv, W1, b1):
  return pl.pallas_call(
      _tc_layer1_body,
      out_shape=jax.ShapeDtypeStruct((N, H), jnp.float32),
  )(parts, xs, dinv, W1, b1)


_RB = 2000


def _tc_layer_body(q_ref, hs_ref, dinv_ref, w_ref, b_ref, out_ref):
  ph = dinv_ref[...] * (q_ref[0] + q_ref[1] + hs_ref[...])
  h = jax.nn.relu(
      jnp.dot(ph, w_ref[...], preferred_element_type=jnp.float32)
      + b_ref[...][None, :])
  out_ref[...] = dinv_ref[...] * h


def _tc_layer(q, hs, dinv, W, b):
  return pl.pallas_call(
      _tc_layer_body,
      grid=(N // _RB,),
      in_specs=[
          pl.BlockSpec((NC, _RB, H), lambda i: (0, i, 0)),
          pl.BlockSpec((_RB, H), lambda i: (i, 0)),
          pl.BlockSpec((_RB, 1), lambda i: (i, 0)),
          pl.BlockSpec((H, H), lambda i: (0, 0)),
          pl.BlockSpec((H,), lambda i: (0,)),
      ],
      out_specs=pl.BlockSpec((_RB, H), lambda i: (i, 0)),
      out_shape=jax.ShapeDtypeStruct((N, H), jnp.float32),
  )(q, hs, dinv, W, b)


def _tc_layer34_body(q_ref, hs_ref, dinv_ref, w3_ref, b3_ref, w4_ref, ts_ref):
  ph = dinv_ref[...] * (q_ref[0] + q_ref[1] + hs_ref[...])
  h3 = jax.nn.relu(
      jnp.dot(ph, w3_ref[...], preferred_element_type=jnp.float32)
      + b3_ref[...][None, :])
  ts_ref[...] = dinv_ref[...] * jnp.dot(
      h3, w4_ref[...], preferred_element_type=jnp.float32)


def _tc_layer34(q, hs, dinv, W3, b3, W4):
  return pl.pallas_call(
      _tc_layer34_body,
      grid=(N // _RB,),
      in_specs=[
          pl.BlockSpec((NC, _RB, H), lambda i: (0, i, 0)),
          pl.BlockSpec((_RB, H), lambda i: (i, 0)),
          pl.BlockSpec((_RB, 1), lambda i: (i, 0)),
          pl.BlockSpec((H, H), lambda i: (0, 0)),
          pl.BlockSpec((H,), lambda i: (0,)),
          pl.BlockSpec((H, 1), lambda i: (0, 0)),
      ],
      out_specs=pl.BlockSpec((_RB, 1), lambda i: (i, 0)),
      out_shape=jax.ShapeDtypeStruct((N, 1), jnp.float32),
  )(q, hs, dinv, W3, b3, W4)


def _tc_final_body(parts_ref, ts_ref, dinv_ref, b4_ref, out_ref):
  o = jnp.sum(parts_ref[...], axis=0)[:, None]
  out_ref[...] = dinv_ref[...] * (o + ts_ref[...]) + b4_ref[0]


def _tc_final(parts, ts, dinv, b4):
  return pl.pallas_call(
      _tc_final_body,
      out_shape=jax.ShapeDtypeStruct((N, 1), jnp.float32),
  )(parts, ts, dinv, b4)


def _pack_bf16(hs):
  lob = lax.bitcast_convert_type(
      hs[:, :H // 2].astype(jnp.bfloat16), jnp.uint16).astype(jnp.uint32)
  hib = lax.bitcast_convert_type(
      hs[:, H // 2:].astype(jnp.bfloat16), jnp.uint16).astype(jnp.uint32)
  return lax.bitcast_convert_type((hib << 16) | lob, jnp.int32)


def _pack_bf16_pad(hs):
  return jnp.pad(_pack_bf16(hs), ((0, NPAD - N), (0, 0)))


def kernel(x, edge_index, edge_weight, W1, b1, W2, b2, W3, b3, W4, b4):
  src = edge_index[0]
  dst = edge_index[1]
  w = edge_weight

  degp = _sc_deg_kernel()(dst, w)
  dinv, xs = _tc_deg(degp, x)
  pparts = _sc_sprop_kernel()(src, dst, w, xs[:, 0])
  hs1 = _tc_layer1(pparts, xs, dinv, W1, b1)
  pad = ((0, 0), (0, EPTP - EPT))
  srcp = jnp.pad(src.reshape(NW, EPT), pad)
  dstp = jnp.pad(dst.reshape(NW, EPT), pad, constant_values=NPAD - 1)
  wp = jnp.pad(w.reshape(NW, EPT), pad)
  packed = (srcp << 16) | dstp
  wbits = lax.bitcast_convert_type(wp, jnp.int32)
  e4 = jnp.stack([packed.reshape(NW, NCHT, CHT),
                  wbits.reshape(NW, NCHT, CHT)], axis=2)
  q2 = _sc_hprop_kernel()(e4, _pack_bf16_pad(hs1))
  hs2 = _tc_layer(q2, hs1, dinv, W2, b2)
  q3 = _sc_hprop_kernel()(e4, _pack_bf16_pad(hs2))
  ts = _tc_layer34(q3, hs2, dinv, W3, b3, W4)
  oparts = _sc_sprop_kernel()(src, dst, w, ts[:, 0])
  return _tc_final(oparts, ts, dinv, b4)
